# Initial kernel scaffold; baseline (speedup 1.0000x reference)
import jax
import jax.numpy as jnp
from jax.experimental import pallas as pl


def kernel(input, embedding_table):
    raise NotImplementedError("write your pallas kernel here")


# speedup vs baseline: 1.7855x; 1.7855x over previous
import functools

import jax
import jax.numpy as jnp
from jax import lax
from jax.experimental import pallas as pl
from jax.experimental.pallas import tpu as pltpu
from jax.experimental.pallas import tpu_sc as plsc

HIST = 200
BATCH = 4096
H = 128
NC = 2
NS = 16
NW = NC * NS
CH = 128
NCHUNK = BATCH // CH
LANES = 16
NV = H // LANES

_mesh = plsc.VectorSubcoreMesh(
    core_axis_name="c", subcore_axis_name="s", num_cores=NC, num_subcores=NS
)


@functools.partial(
    pl.kernel,
    mesh=_mesh,
    out_type=jax.ShapeDtypeStruct((HIST, H), jnp.float32),
    scratch_types=[
        pltpu.VMEM((NCHUNK, CH), jnp.int32),
        pltpu.VMEM((CH, H), jnp.float32),
        pltpu.VMEM((CH, H), jnp.float32),
        pltpu.VMEM((H,), jnp.float32),
        pltpu.SemaphoreType.DMA,
        pltpu.SemaphoreType.DMA,
    ],
)
def _embed_sum(idx_hbm, table_hbm, out_hbm, idx_v, buf_a, buf_b, acc_v,
               sem_a, sem_b):
    wid = lax.axis_index("s") * NC + lax.axis_index("c")
    start = wid * 6 + jnp.minimum(wid, 8)
    cnt = jnp.where(wid < 8, 7, 6)

    def accum(buf, acc):
        def row(r, acc):
            return tuple(
                acc[h] + buf[r, pl.ds(LANES * h, LANES)] for h in range(NV)
            )
        return lax.fori_loop(0, CH, row, acc)

    def do_position(p, carry):
        l = start + p
        pltpu.sync_copy(idx_hbm.at[l], idx_v)
        pltpu.async_copy(table_hbm.at[idx_v.at[0]], buf_a, sem_a)
        acc0 = (jnp.zeros((LANES,), jnp.float32),) * NV

        def two_chunks(j2, acc):
            j = 2 * j2
            pltpu.async_copy(table_hbm.at[idx_v.at[j + 1]], buf_b, sem_b)
            pltpu.make_async_copy(table_hbm.at[idx_v.at[j]], buf_a, sem_a).wait()
            acc = accum(buf_a, acc)

            @pl.when(j + 2 < NCHUNK)
            def _():
                pltpu.async_copy(table_hbm.at[idx_v.at[j + 2]], buf_a, sem_a)

            pltpu.make_async_copy(
                table_hbm.at[idx_v.at[j + 1]], buf_b, sem_b
            ).wait()
            acc = accum(buf_b, acc)
            return acc

        acc = lax.fori_loop(0, NCHUNK // 2, two_chunks, acc0)
        for h in range(NV):
            acc_v[pl.ds(LANES * h, LANES)] = acc[h]
        pltpu.sync_copy(acc_v, out_hbm.at[l])
        return carry

    lax.fori_loop(0, cnt, do_position, 0)


def kernel(input, embedding_table):
    idx = jnp.transpose(input).reshape(HIST, NCHUNK, CH).astype(jnp.int32)
    out = _embed_sum(idx, embedding_table)
    return out.reshape(1, HIST * H)


# speedup vs baseline: 1.9527x; 1.0937x over previous
import functools

import jax
import jax.numpy as jnp
from jax import lax
from jax.experimental import pallas as pl
from jax.experimental.pallas import tpu as pltpu
from jax.experimental.pallas import tpu_sc as plsc

HIST = 200
BATCH = 4096
H = 128
NC = 2
NS = 16
CH = 128
NCHUNK = BATCH // CH
LANES = 16
NV = H // LANES
GSZ = 4
PPG = 25
FULL = 6

_mesh = plsc.VectorSubcoreMesh(
    core_axis_name="c", subcore_axis_name="s", num_cores=NC, num_subcores=NS
)


@functools.partial(
    pl.kernel,
    mesh=_mesh,
    out_type=jax.ShapeDtypeStruct((HIST, H), jnp.float32),
    scratch_types=[
        pltpu.VMEM((NCHUNK, CH), jnp.int32),
        pltpu.VMEM((CH, H), jnp.float32),
        pltpu.VMEM((CH, H), jnp.float32),
        pltpu.VMEM((H,), jnp.float32),
        pltpu.VMEM((GSZ, H), jnp.float32),
        pltpu.VMEM_SHARED((GSZ, GSZ, H), jnp.float32),
        pltpu.SemaphoreType.DMA,
        pltpu.SemaphoreType.DMA,
    ],
)
def _embed_sum(idx_hbm, table_hbm, out_hbm, idx_v, buf_a, buf_b, acc_v,
               red_v, part_sh, sem_a, sem_b):
    c = lax.axis_index("c")
    s = lax.axis_index("s")
    sg = s // GSZ
    r = s % GSZ
    base = (c * GSZ + sg) * PPG

    def accum(buf, acc):
        def row(rr, acc):
            return tuple(
                acc[h] + buf[rr, pl.ds(LANES * h, LANES)] for h in range(NV)
            )
        return lax.fori_loop(0, CH, row, acc)

    def sum_chunks(j0, n2):
        pltpu.async_copy(table_hbm.at[idx_v.at[j0]], buf_a, sem_a)
        lim = j0 + 2 * n2
        acc0 = (jnp.zeros((LANES,), jnp.float32),) * NV

        def two_chunks(i, acc):
            j = j0 + 2 * i
            pltpu.async_copy(table_hbm.at[idx_v.at[j + 1]], buf_b, sem_b)
            pltpu.make_async_copy(table_hbm.at[idx_v.at[j]], buf_a, sem_a).wait()
            acc = accum(buf_a, acc)

            @pl.when(j + 2 < lim)
            def _():
                pltpu.async_copy(table_hbm.at[idx_v.at[j + 2]], buf_a, sem_a)

            pltpu.make_async_copy(
                table_hbm.at[idx_v.at[j + 1]], buf_b, sem_b
            ).wait()
            acc = accum(buf_b, acc)
            return acc

        return lax.fori_loop(0, n2, two_chunks, acc0)

    def store_acc(acc):
        for h in range(NV):
            acc_v[pl.ds(LANES * h, LANES)] = acc[h]

    def do_position(p, carry):
        l = base + r * FULL + p
        pltpu.sync_copy(idx_hbm.at[l], idx_v)
        store_acc(sum_chunks(0, NCHUNK // 2))
        pltpu.sync_copy(acc_v, out_hbm.at[l])
        return carry

    lax.fori_loop(0, FULL, do_position, 0)

    ps = base + GSZ * FULL
    pltpu.sync_copy(idx_hbm.at[ps], idx_v)
    store_acc(sum_chunks(r * (NCHUNK // GSZ), NCHUNK // (2 * GSZ)))
    pltpu.sync_copy(acc_v, part_sh.at[sg, r])
    plsc.subcore_barrier()

    @pl.when(r == 0)
    def _():
        pltpu.sync_copy(part_sh.at[sg], red_v)
        acc = tuple(
            red_v[0, pl.ds(LANES * h, LANES)] for h in range(NV)
        )
        for q in range(1, GSZ):
            acc = tuple(
                acc[h] + red_v[q, pl.ds(LANES * h, LANES)] for h in range(NV)
            )
        store_acc(acc)
        pltpu.sync_copy(acc_v, out_hbm.at[ps])


def kernel(input, embedding_table):
    idx = jnp.transpose(input).reshape(HIST, NCHUNK, CH).astype(jnp.int32)
    out = _embed_sum(idx, embedding_table)
    return out.reshape(1, HIST * H)


# speedup vs baseline: 2.0552x; 1.0525x over previous
import functools

import jax
import jax.numpy as jnp
from jax import lax
from jax.experimental import pallas as pl
from jax.experimental.pallas import tpu as pltpu
from jax.experimental.pallas import tpu_sc as plsc

HIST = 200
BATCH = 4096
H = 128
NC = 2
NS = 16
CH = 128
NCHUNK = BATCH // CH
LANES = 16
NV = H // LANES
GSZ = 4
PPG = 25
FULL = 6
QCH = NCHUNK // GSZ
TCH = FULL * NCHUNK + QCH
TPAIR = TCH // 2

_mesh = plsc.VectorSubcoreMesh(
    core_axis_name="c", subcore_axis_name="s", num_cores=NC, num_subcores=NS
)


@functools.partial(
    pl.kernel,
    mesh=_mesh,
    out_type=jax.ShapeDtypeStruct((HIST, H), jnp.float32),
    scratch_types=[
        pltpu.VMEM((TCH, CH), jnp.int32),
        pltpu.VMEM((CH, H), jnp.float32),
        pltpu.VMEM((CH, H), jnp.float32),
        pltpu.VMEM((H,), jnp.float32),
        pltpu.VMEM((GSZ, H), jnp.float32),
        pltpu.VMEM_SHARED((GSZ, GSZ, H), jnp.float32),
        pltpu.SemaphoreType.DMA,
        pltpu.SemaphoreType.DMA,
        pltpu.SemaphoreType.DMA,
    ],
)
def _embed_sum(idx_hbm, table_hbm, out_hbm, idx_v, buf_a, buf_b, acc_v,
               red_v, part_sh, sem_a, sem_b, sem_i):
    c = lax.axis_index("c")
    s = lax.axis_index("s")
    sg = s // GSZ
    r = s % GSZ
    base = (c * GSZ + sg) * PPG
    ps = base + GSZ * FULL

    pltpu.sync_copy(idx_hbm.at[base + r * FULL], idx_v.at[pl.ds(0, NCHUNK)])
    pltpu.async_copy(table_hbm.at[idx_v.at[0]], buf_a, sem_a)
    for p in range(1, FULL):
        pltpu.async_copy(
            idx_hbm.at[base + r * FULL + p],
            idx_v.at[pl.ds(p * NCHUNK, NCHUNK)],
            sem_i,
        )
    pltpu.async_copy(
        idx_hbm.at[ps, pl.ds(r * QCH, QCH)],
        idx_v.at[pl.ds(FULL * NCHUNK, QCH)],
        sem_i,
    )
    for p in range(1, FULL):
        pltpu.make_async_copy(
            idx_hbm.at[base + r * FULL + p],
            idx_v.at[pl.ds(p * NCHUNK, NCHUNK)],
            sem_i,
        ).wait()
    pltpu.make_async_copy(
        idx_hbm.at[ps, pl.ds(r * QCH, QCH)],
        idx_v.at[pl.ds(FULL * NCHUNK, QCH)],
        sem_i,
    ).wait()

    def accum(buf, acc):
        def row(rr, acc):
            return tuple(
                acc[h] + buf[rr, pl.ds(LANES * h, LANES)] for h in range(NV)
            )
        return lax.fori_loop(0, CH, row, acc)

    def run_pairs(t0, n):
        acc0 = (jnp.zeros((LANES,), jnp.float32),) * NV

        def body(i, acc):
            j = 2 * (t0 + i)
            pltpu.async_copy(table_hbm.at[idx_v.at[j + 1]], buf_b, sem_b)
            pltpu.make_async_copy(table_hbm.at[idx_v.at[j]], buf_a, sem_a).wait()
            acc = accum(buf_a, acc)

            @pl.when(j + 2 < TCH)
            def _():
                pltpu.async_copy(table_hbm.at[idx_v.at[j + 2]], buf_a, sem_a)

            pltpu.make_async_copy(
                table_hbm.at[idx_v.at[j + 1]], buf_b, sem_b
            ).wait()
            acc = accum(buf_b, acc)
            return acc

        return lax.fori_loop(0, n, body, acc0)

    def store_acc(acc):
        for h in range(NV):
            acc_v[pl.ds(LANES * h, LANES)] = acc[h]

    def do_position(p, carry):
        acc = run_pairs(p * (NCHUNK // 2), NCHUNK // 2)
        store_acc(acc)
        pltpu.sync_copy(acc_v, out_hbm.at[base + r * FULL + p])
        return carry

    lax.fori_loop(0, FULL, do_position, 0)

    store_acc(run_pairs(FULL * (NCHUNK // 2), QCH // 2))
    pltpu.sync_copy(acc_v, part_sh.at[sg, r])
    plsc.subcore_barrier()

    @pl.when(r == 0)
    def _():
        pltpu.sync_copy(part_sh.at[sg], red_v)
        acc = tuple(red_v[0, pl.ds(LANES * h, LANES)] for h in range(NV))
        for q in range(1, GSZ):
            acc = tuple(
                acc[h] + red_v[q, pl.ds(LANES * h, LANES)] for h in range(NV)
            )
        store_acc(acc)
        pltpu.sync_copy(acc_v, out_hbm.at[ps])


def kernel(input, embedding_table):
    idx = jnp.transpose(input).reshape(HIST, NCHUNK, CH).astype(jnp.int32)
    out = _embed_sum(idx, embedding_table)
    return out.reshape(1, HIST * H)


# speedup vs baseline: 2.0567x; 1.0007x over previous
import functools

import jax
import jax.numpy as jnp
from jax import lax
from jax.experimental import pallas as pl
from jax.experimental.pallas import tpu as pltpu
from jax.experimental.pallas import tpu_sc as plsc

HIST = 200
BATCH = 4096
H = 128
NC = 2
NS = 16
CH = 128
NCHUNK = BATCH // CH
LANES = 16
NV = H // LANES
GSZ = 4
PPG = 25
FULL = 6
QCH = NCHUNK // GSZ
TCH = FULL * NCHUNK + QCH
TPAIR = TCH // 2

_mesh = plsc.VectorSubcoreMesh(
    core_axis_name="c", subcore_axis_name="s", num_cores=NC, num_subcores=NS
)


@functools.partial(
    pl.kernel,
    mesh=_mesh,
    out_type=jax.ShapeDtypeStruct((HIST, H), jnp.float32),
    scratch_types=[
        pltpu.VMEM((TCH, CH), jnp.int32),
        pltpu.VMEM((CH, H), jnp.float32),
        pltpu.VMEM((CH, H), jnp.float32),
        pltpu.VMEM((H,), jnp.float32),
        pltpu.VMEM((GSZ, H), jnp.float32),
        pltpu.VMEM_SHARED((GSZ, GSZ, H), jnp.float32),
        pltpu.SemaphoreType.DMA,
        pltpu.SemaphoreType.DMA,
        pltpu.SemaphoreType.DMA,
    ],
)
def _embed_sum(idx_hbm, table_hbm, out_hbm, idx_v, buf_a, buf_b, acc_v,
               red_v, part_sh, sem_a, sem_b, sem_i):
    c = lax.axis_index("c")
    s = lax.axis_index("s")
    sg = s // GSZ
    r = s % GSZ
    base = (c * GSZ + sg) * PPG
    ps = base + GSZ * FULL

    pltpu.sync_copy(idx_hbm.at[base + r * FULL], idx_v.at[pl.ds(0, NCHUNK)])
    pltpu.async_copy(table_hbm.at[idx_v.at[0]], buf_a, sem_a)
    for p in range(1, FULL):
        pltpu.async_copy(
            idx_hbm.at[base + r * FULL + p],
            idx_v.at[pl.ds(p * NCHUNK, NCHUNK)],
            sem_i,
        )
    pltpu.async_copy(
        idx_hbm.at[ps, pl.ds(r * QCH, QCH)],
        idx_v.at[pl.ds(FULL * NCHUNK, QCH)],
        sem_i,
    )
    for p in range(1, FULL):
        pltpu.make_async_copy(
            idx_hbm.at[base + r * FULL + p],
            idx_v.at[pl.ds(p * NCHUNK, NCHUNK)],
            sem_i,
        ).wait()
    pltpu.make_async_copy(
        idx_hbm.at[ps, pl.ds(r * QCH, QCH)],
        idx_v.at[pl.ds(FULL * NCHUNK, QCH)],
        sem_i,
    ).wait()

    def accum(buf, acc):
        def rows(rr, acc):
            r0 = 2 * rr
            acc = tuple(
                acc[h] + buf[r0, pl.ds(LANES * h, LANES)] for h in range(NV)
            )
            return tuple(
                acc[h] + buf[r0 + 1, pl.ds(LANES * h, LANES)]
                for h in range(NV)
            )
        return lax.fori_loop(0, CH // 2, rows, acc)

    def run_pairs(t0, n):
        acc0 = (jnp.zeros((LANES,), jnp.float32),) * NV

        def body(i, acc):
            j = 2 * (t0 + i)
            pltpu.async_copy(table_hbm.at[idx_v.at[j + 1]], buf_b, sem_b)
            pltpu.make_async_copy(table_hbm.at[idx_v.at[j]], buf_a, sem_a).wait()
            acc = accum(buf_a, acc)

            @pl.when(j + 2 < TCH)
            def _():
                pltpu.async_copy(table_hbm.at[idx_v.at[j + 2]], buf_a, sem_a)

            pltpu.make_async_copy(
                table_hbm.at[idx_v.at[j + 1]], buf_b, sem_b
            ).wait()
            acc = accum(buf_b, acc)
            return acc

        return lax.fori_loop(0, n, body, acc0)

    def store_acc(acc):
        for h in range(NV):
            acc_v[pl.ds(LANES * h, LANES)] = acc[h]

    def do_position(p, carry):
        acc = run_pairs(p * (NCHUNK // 2), NCHUNK // 2)
        store_acc(acc)
        pltpu.sync_copy(acc_v, out_hbm.at[base + r * FULL + p])
        return carry

    lax.fori_loop(0, FULL, do_position, 0)

    store_acc(run_pairs(FULL * (NCHUNK // 2), QCH // 2))
    pltpu.sync_copy(acc_v, part_sh.at[sg, r])
    plsc.subcore_barrier()

    @pl.when(r == 0)
    def _():
        pltpu.sync_copy(part_sh.at[sg], red_v)
        acc = tuple(red_v[0, pl.ds(LANES * h, LANES)] for h in range(NV))
        for q in range(1, GSZ):
            acc = tuple(
                acc[h] + red_v[q, pl.ds(LANES * h, LANES)] for h in range(NV)
            )
        store_acc(acc)
        pltpu.sync_copy(acc_v, out_hbm.at[ps])


def kernel(input, embedding_table):
    idx = jnp.transpose(input).reshape(HIST, NCHUNK, CH).astype(jnp.int32)
    out = _embed_sum(idx, embedding_table)
    return out.reshape(1, HIST * H)


# speedup vs baseline: 2.6915x; 1.3087x over previous
import functools

import jax
import jax.numpy as jnp
from jax import lax
from jax.experimental import pallas as pl
from jax.experimental.pallas import tpu as pltpu
from jax.experimental.pallas import tpu_sc as plsc

HIST = 200
BATCH = 4096
H = 128
NC = 2
NS = 16
CH = 128
NCHUNK = BATCH // CH
LANES = 16
NV = H // LANES
GSZ = 4
PPG = 25
FULL = 6
QCH = NCHUNK // GSZ
TCH = FULL * NCHUNK + QCH
NBUF = 3
NTRIP = (TCH - 2) // NBUF

_mesh = plsc.VectorSubcoreMesh(
    core_axis_name="c", subcore_axis_name="s", num_cores=NC, num_subcores=NS
)


@functools.partial(
    pl.kernel,
    mesh=_mesh,
    out_type=jax.ShapeDtypeStruct((HIST, H), jnp.float32),
    scratch_types=[
        pltpu.VMEM((TCH, CH), jnp.int32),
        pltpu.VMEM((CH, H), jnp.float32),
        pltpu.VMEM((CH, H), jnp.float32),
        pltpu.VMEM((CH, H), jnp.float32),
        pltpu.VMEM((FULL + 1, H), jnp.float32),
        pltpu.VMEM((GSZ, H), jnp.float32),
        pltpu.VMEM_SHARED((GSZ, GSZ, H), jnp.float32),
        pltpu.SemaphoreType.DMA,
        pltpu.SemaphoreType.DMA,
        pltpu.SemaphoreType.DMA,
        pltpu.SemaphoreType.DMA,
    ],
)
def _embed_sum(idx_hbm, table_hbm, out_hbm, idx_v, buf_a, buf_b, buf_c,
               acc_v, red_v, part_sh, sem_a, sem_b, sem_c, sem_o):
    c = lax.axis_index("c")
    s = lax.axis_index("s")
    sg = s // GSZ
    r = s % GSZ
    base = (c * GSZ + sg) * PPG
    ps = base + GSZ * FULL

    pltpu.sync_copy(idx_hbm.at[base + r * FULL], idx_v.at[pl.ds(0, NCHUNK)])
    pltpu.async_copy(table_hbm.at[idx_v.at[0]], buf_a, sem_a)
    pltpu.async_copy(table_hbm.at[idx_v.at[1]], buf_b, sem_b)
    pltpu.async_copy(table_hbm.at[idx_v.at[2]], buf_c, sem_c)
    for p in range(1, FULL):
        pltpu.async_copy(
            idx_hbm.at[base + r * FULL + p],
            idx_v.at[pl.ds(p * NCHUNK, NCHUNK)],
            sem_o,
        )
    pltpu.async_copy(
        idx_hbm.at[ps, pl.ds(r * QCH, QCH)],
        idx_v.at[pl.ds(FULL * NCHUNK, QCH)],
        sem_o,
    )
    for p in range(1, FULL):
        pltpu.make_async_copy(
            idx_hbm.at[base + r * FULL + p],
            idx_v.at[pl.ds(p * NCHUNK, NCHUNK)],
            sem_o,
        ).wait()
    pltpu.make_async_copy(
        idx_hbm.at[ps, pl.ds(r * QCH, QCH)],
        idx_v.at[pl.ds(FULL * NCHUNK, QCH)],
        sem_o,
    ).wait()

    def accum(buf, acc):
        def rows(rr, acc):
            r0 = 2 * rr
            acc = tuple(
                acc[h] + buf[r0, pl.ds(LANES * h, LANES)] for h in range(NV)
            )
            return tuple(
                acc[h] + buf[r0 + 1, pl.ds(LANES * h, LANES)]
                for h in range(NV)
            )
        return lax.fori_loop(0, CH // 2, rows, acc)

    zero = jnp.zeros((LANES,), jnp.float32)

    def chunk_step(j, buf, sem, acc):
        pltpu.make_async_copy(table_hbm.at[idx_v.at[j]], buf, sem).wait()
        acc = accum(buf, acc)

        @pl.when(j + NBUF < TCH)
        def _():
            pltpu.async_copy(table_hbm.at[idx_v.at[j + NBUF]], buf, sem)

        done = lax.rem(j + 1, NCHUNK) == 0
        p = lax.div(j + 1, NCHUNK) - 1

        @pl.when(done)
        def _():
            for h in range(NV):
                acc_v[p, pl.ds(LANES * h, LANES)] = acc[h]
            pltpu.async_copy(acc_v.at[p], out_hbm.at[base + r * FULL + p],
                             sem_o)

        return tuple(jnp.where(done, zero, a) for a in acc)

    def ring(i, acc):
        j = NBUF * i
        acc = chunk_step(j, buf_a, sem_a, acc)
        acc = chunk_step(j + 1, buf_b, sem_b, acc)
        acc = chunk_step(j + 2, buf_c, sem_c, acc)
        return acc

    acc = lax.fori_loop(0, NTRIP, ring, (zero,) * NV)
    acc = chunk_step(NBUF * NTRIP, buf_a, sem_a, acc)
    acc = chunk_step(NBUF * NTRIP + 1, buf_b, sem_b, acc)

    for h in range(NV):
        acc_v[FULL, pl.ds(LANES * h, LANES)] = acc[h]
    pltpu.sync_copy(acc_v.at[FULL], part_sh.at[sg, r])

    for p in range(FULL):
        pltpu.make_async_copy(acc_v.at[p], out_hbm.at[base + r * FULL + p],
                              sem_o).wait()
    plsc.subcore_barrier()

    @pl.when(r == 0)
    def _():
        pltpu.sync_copy(part_sh.at[sg], red_v)
        facc = tuple(red_v[0, pl.ds(LANES * h, LANES)] for h in range(NV))
        for q in range(1, GSZ):
            facc = tuple(
                facc[h] + red_v[q, pl.ds(LANES * h, LANES)] for h in range(NV)
            )
        for h in range(NV):
            acc_v[FULL, pl.ds(LANES * h, LANES)] = facc[h]
        pltpu.sync_copy(acc_v.at[FULL], out_hbm.at[ps])


def kernel(input, embedding_table):
    idx = jnp.transpose(input).reshape(HIST, NCHUNK, CH).astype(jnp.int32)
    out = _embed_sum(idx, embedding_table)
    return out.reshape(1, HIST * H)


# speedup vs baseline: 2.9041x; 1.0790x over previous
import functools

import jax
import jax.numpy as jnp
from jax import lax
from jax.experimental import pallas as pl
from jax.experimental.pallas import tpu as pltpu
from jax.experimental.pallas import tpu_sc as plsc

HIST = 200
BATCH = 4096
H = 128
NC = 2
NS = 16
CH = 128
NCHUNK = BATCH // CH
LANES = 16
NV = H // LANES
GSZ = 4
PPG = 25
FULL = 6
QCH = NCHUNK // GSZ
TCH = FULL * NCHUNK + QCH
NBUF = 4
NTRIP = TCH // NBUF

_mesh = plsc.VectorSubcoreMesh(
    core_axis_name="c", subcore_axis_name="s", num_cores=NC, num_subcores=NS
)


@functools.partial(
    pl.kernel,
    mesh=_mesh,
    out_type=jax.ShapeDtypeStruct((HIST, H), jnp.float32),
    scratch_types=[
        pltpu.VMEM((TCH, CH), jnp.int32),
        pltpu.VMEM((CH, H), jnp.float32),
        pltpu.VMEM((CH, H), jnp.float32),
        pltpu.VMEM((CH, H), jnp.float32),
        pltpu.VMEM((CH, H), jnp.float32),
        pltpu.VMEM((FULL + 1, H), jnp.float32),
        pltpu.VMEM((GSZ, H), jnp.float32),
        pltpu.VMEM_SHARED((GSZ, GSZ, H), jnp.float32),
        pltpu.SemaphoreType.DMA,
        pltpu.SemaphoreType.DMA,
        pltpu.SemaphoreType.DMA,
        pltpu.SemaphoreType.DMA,
        pltpu.SemaphoreType.DMA,
    ],
)
def _embed_sum(idx_hbm, table_hbm, out_hbm, idx_v, buf_a, buf_b, buf_c,
               buf_d, acc_v, red_v, part_sh, sem_a, sem_b, sem_c, sem_d,
               sem_o):
    c = lax.axis_index("c")
    s = lax.axis_index("s")
    sg = s // GSZ
    r = s % GSZ
    base = (c * GSZ + sg) * PPG
    ps = base + GSZ * FULL

    pltpu.sync_copy(idx_hbm.at[base + r * FULL], idx_v.at[pl.ds(0, NCHUNK)])
    pltpu.async_copy(table_hbm.at[idx_v.at[0]], buf_a, sem_a)
    pltpu.async_copy(table_hbm.at[idx_v.at[1]], buf_b, sem_b)
    pltpu.async_copy(table_hbm.at[idx_v.at[2]], buf_c, sem_c)
    pltpu.async_copy(table_hbm.at[idx_v.at[3]], buf_d, sem_d)
    for p in range(1, FULL):
        pltpu.async_copy(
            idx_hbm.at[base + r * FULL + p],
            idx_v.at[pl.ds(p * NCHUNK, NCHUNK)],
            sem_o,
        )
    pltpu.async_copy(
        idx_hbm.at[ps, pl.ds(r * QCH, QCH)],
        idx_v.at[pl.ds(FULL * NCHUNK, QCH)],
        sem_o,
    )
    for p in range(1, FULL):
        pltpu.make_async_copy(
            idx_hbm.at[base + r * FULL + p],
            idx_v.at[pl.ds(p * NCHUNK, NCHUNK)],
            sem_o,
        ).wait()
    pltpu.make_async_copy(
        idx_hbm.at[ps, pl.ds(r * QCH, QCH)],
        idx_v.at[pl.ds(FULL * NCHUNK, QCH)],
        sem_o,
    ).wait()

    def accum(buf, acc):
        def rows(rr, acc):
            r0 = 2 * rr
            acc = tuple(
                acc[h] + buf[r0, pl.ds(LANES * h, LANES)] for h in range(NV)
            )
            return tuple(
                acc[h] + buf[r0 + 1, pl.ds(LANES * h, LANES)]
                for h in range(NV)
            )
        return lax.fori_loop(0, CH // 2, rows, acc)

    zero = jnp.zeros((LANES,), jnp.float32)

    def chunk_step(j, buf, sem, acc):
        pltpu.make_async_copy(table_hbm.at[idx_v.at[j]], buf, sem).wait()
        acc = accum(buf, acc)

        @pl.when(j + NBUF < TCH)
        def _():
            pltpu.async_copy(table_hbm.at[idx_v.at[j + NBUF]], buf, sem)

        done = lax.rem(j + 1, NCHUNK) == 0
        p = lax.div(j + 1, NCHUNK) - 1

        @pl.when(done)
        def _():
            for h in range(NV):
                acc_v[p, pl.ds(LANES * h, LANES)] = acc[h]
            pltpu.async_copy(acc_v.at[p], out_hbm.at[base + r * FULL + p],
                             sem_o)

        return tuple(jnp.where(done, zero, a) for a in acc)

    def ring(i, acc):
        j = NBUF * i
        acc = chunk_step(j, buf_a, sem_a, acc)
        acc = chunk_step(j + 1, buf_b, sem_b, acc)
        acc = chunk_step(j + 2, buf_c, sem_c, acc)
        acc = chunk_step(j + 3, buf_d, sem_d, acc)
        return acc

    acc = lax.fori_loop(0, NTRIP, ring, (zero,) * NV)

    for h in range(NV):
        acc_v[FULL, pl.ds(LANES * h, LANES)] = acc[h]
    pltpu.sync_copy(acc_v.at[FULL], part_sh.at[sg, r])

    for p in range(FULL):
        pltpu.make_async_copy(acc_v.at[p], out_hbm.at[base + r * FULL + p],
                              sem_o).wait()
    plsc.subcore_barrier()

    @pl.when(r == 0)
    def _():
        pltpu.sync_copy(part_sh.at[sg], red_v)
        facc = tuple(red_v[0, pl.ds(LANES * h, LANES)] for h in range(NV))
        for q in range(1, GSZ):
            facc = tuple(
                facc[h] + red_v[q, pl.ds(LANES * h, LANES)] for h in range(NV)
            )
        for h in range(NV):
            acc_v[FULL, pl.ds(LANES * h, LANES)] = facc[h]
        pltpu.sync_copy(acc_v.at[FULL], out_hbm.at[ps])


def kernel(input, embedding_table):
    idx = jnp.transpose(input).reshape(HIST, NCHUNK, CH).astype(jnp.int32)
    out = _embed_sum(idx, embedding_table)
    return out.reshape(1, HIST * H)
